# Initial kernel scaffold; baseline (speedup 1.0000x reference)
import jax
import jax.numpy as jnp
from jax.experimental import pallas as pl


def kernel(x, theta1, bias1, theta2, bias2):
    raise NotImplementedError("write your pallas kernel here")


# speedup vs baseline: 4.2516x; 4.2516x over previous
import jax
import jax.numpy as jnp
from jax import lax
from jax.experimental import pallas as pl

K = 8
EPS = 0.1
RB = 256


def _normalize_body(x_ref, o_ref):
    x = x_ref[...]
    nrm = jnp.sqrt(jnp.sum(x * x, axis=1, keepdims=True))
    o_ref[...] = x / jnp.maximum(nrm, 1e-12)


def _graph_body(xb_ref, xall_ref, idx_ref, w_ref):
    n = xall_ref.shape[0]
    s = lax.dot_general(xb_ref[...], xall_ref[...], (((1,), (1,)), ((), ())),
                        preferred_element_type=jnp.float32)
    cols = lax.broadcasted_iota(jnp.int32, (RB, n), 1)
    kcols = lax.broadcasted_iota(jnp.int32, (RB, K), 1)
    idx_out = jnp.zeros((RB, K), jnp.int32)
    w_out = jnp.zeros((RB, K), jnp.float32)
    for k in range(K):
        m = jnp.max(s, axis=1, keepdims=True)
        amin = jnp.min(jnp.where(s == m, cols, n), axis=1, keepdims=True)
        idx_out = jnp.where(kcols == k, amin, idx_out)
        w_out = jnp.where(kcols == k, (m > EPS).astype(jnp.float32), w_out)
        s = jnp.where(cols == amin, -jnp.inf, s)
    idx_ref[...] = idx_out
    w_ref[...] = w_out


def _matmul_body(a_ref, b_ref, o_ref):
    o_ref[...] = jnp.dot(a_ref[...], b_ref[...],
                         preferred_element_type=jnp.float32)


def _edge_body(idxT_ref, wT_ref, xl_ref, ef_ref):
    n = xl_ref.shape[0]
    j = pl.program_id(1)
    ji = j * RB + lax.broadcasted_iota(jnp.int32, (RB, 1), 0)
    ht = jnp.zeros((RB, n), jnp.float32)
    for k in range(K):
        rowi = idxT_ref[k:k + 1, :]
        roww = wT_ref[k:k + 1, :]
        ht = ht + jnp.where(rowi == ji, roww, 0.0)
    bdeg = jnp.sum(ht, axis=1, keepdims=True)
    binv = jnp.where(bdeg > 0, 1.0 / bdeg, 0.0)
    ef_ref[...] = jnp.dot(ht, xl_ref[...],
                          preferred_element_type=jnp.float32) * binv


def _node_body(idx_ref, w_ref, ef_ref, b_ref, o_ref, *, act):
    n = ef_ref.shape[0]
    cols = lax.broadcasted_iota(jnp.int32, (RB, n), 1)
    h = jnp.zeros((RB, n), jnp.float32)
    for k in range(K):
        h = h + jnp.where(idx_ref[:, k:k + 1] == cols, w_ref[:, k:k + 1], 0.0)
    ddeg = jnp.sum(w_ref[...], axis=1, keepdims=True)
    dinv = jnp.where(ddeg > 0, 1.0 / ddeg, 0.0)
    out = jnp.dot(h, ef_ref[...],
                  preferred_element_type=jnp.float32) * dinv + b_ref[...]
    o_ref[...] = act(out)


def _matmul(a, b, cb):
    m, kd = a.shape
    _, nd = b.shape
    return pl.pallas_call(
        _matmul_body,
        grid=(nd // cb, m // RB),
        in_specs=[pl.BlockSpec((RB, kd), lambda c, i: (i, 0)),
                  pl.BlockSpec((kd, cb), lambda c, i: (0, c))],
        out_specs=pl.BlockSpec((RB, cb), lambda c, i: (i, c)),
        out_shape=jax.ShapeDtypeStruct((m, nd), jnp.float32),
    )(a, b)


def _edge_agg(idxT, wT, xl, cb):
    n = xl.shape[0]
    w_ = xl.shape[1]
    return pl.pallas_call(
        _edge_body,
        grid=(w_ // cb, n // RB),
        in_specs=[pl.BlockSpec((K, n), lambda c, j: (0, 0)),
                  pl.BlockSpec((K, n), lambda c, j: (0, 0)),
                  pl.BlockSpec((n, cb), lambda c, j: (0, c))],
        out_specs=pl.BlockSpec((RB, cb), lambda c, j: (j, c)),
        out_shape=jax.ShapeDtypeStruct((n, w_), jnp.float32),
    )(idxT, wT, xl)


def _node_agg(idx, w, ef, bias, cb, act):
    import functools
    n = ef.shape[0]
    w_ = ef.shape[1]
    return pl.pallas_call(
        functools.partial(_node_body, act=act),
        grid=(w_ // cb, n // RB),
        in_specs=[pl.BlockSpec((RB, K), lambda c, i: (i, 0)),
                  pl.BlockSpec((RB, K), lambda c, i: (i, 0)),
                  pl.BlockSpec((n, cb), lambda c, i: (0, c)),
                  pl.BlockSpec((1, cb), lambda c, i: (0, c))],
        out_specs=pl.BlockSpec((RB, cb), lambda c, i: (i, c)),
        out_shape=jax.ShapeDtypeStruct((n, w_), jnp.float32),
    )(idx, w, ef, bias.reshape(1, -1))


def kernel(x, theta1, bias1, theta2, bias2):
    n, d_in = x.shape
    hid = theta1.shape[1]
    code = theta2.shape[1]

    xn = pl.pallas_call(
        _normalize_body,
        grid=(n // RB,),
        in_specs=[pl.BlockSpec((RB, d_in), lambda i: (i, 0))],
        out_specs=pl.BlockSpec((RB, d_in), lambda i: (i, 0)),
        out_shape=jax.ShapeDtypeStruct((n, d_in), jnp.float32),
    )(x)

    idx, w = pl.pallas_call(
        _graph_body,
        grid=(n // RB,),
        in_specs=[pl.BlockSpec((RB, d_in), lambda i: (i, 0)),
                  pl.BlockSpec((n, d_in), lambda i: (0, 0))],
        out_specs=[pl.BlockSpec((RB, K), lambda i: (i, 0)),
                   pl.BlockSpec((RB, K), lambda i: (i, 0))],
        out_shape=[jax.ShapeDtypeStruct((n, K), jnp.int32),
                   jax.ShapeDtypeStruct((n, K), jnp.float32)],
    )(xn, xn)

    idxT = idx.T
    wT = w.T

    xl1 = _matmul(x, theta1, 512)
    ef1 = _edge_agg(idxT, wT, xl1, 512)
    out1 = _node_agg(idx, w, ef1, bias1, 512, jax.nn.relu)

    xl2 = _matmul(out1, theta2, code)
    ef2 = _edge_agg(idxT, wT, xl2, code)
    out2 = _node_agg(idx, w, ef2, bias2, code, jnp.tanh)
    return out2
